# Initial kernel scaffold; baseline (speedup 1.0000x reference)
import jax
import jax.numpy as jnp
from jax.experimental import pallas as pl


def kernel(x, edge_index, edge_attr, batch, Wl1, bl1, Wr1, br1, We1, att1, bias1, Wl2, bl2, Wr2, br2, We2, att2, bias2, ln_gamma, ln_beta):
    raise NotImplementedError("write your pallas kernel here")


# speedup vs baseline: 1.0879x; 1.0879x over previous
import functools

import jax
import jax.numpy as jnp
from jax.experimental import pallas as pl
from jax.experimental.pallas import tpu as pltpu

N = 10000
E = 320000
D_IN = 128
D_EDGE = 16
C_OUT = 128
NUM_GRAPHS = 64


def _mm_body(a_ref, w_ref, b_ref, o_ref):
    o_ref[...] = jnp.dot(a_ref[...], w_ref[...],
                         preferred_element_type=jnp.float32) + b_ref[...]


@functools.partial(jax.jit, static_argnames=("block_rows",))
def _mm(a, w, b, block_rows=1000):
    m, k = a.shape
    n = w.shape[1]
    grid = (pl.cdiv(m, block_rows),)
    return pl.pallas_call(
        _mm_body,
        grid=grid,
        in_specs=[
            pl.BlockSpec((block_rows, k), lambda i: (i, 0)),
            pl.BlockSpec((k, n), lambda i: (0, 0)),
            pl.BlockSpec((1, n), lambda i: (0, 0)),
        ],
        out_specs=pl.BlockSpec((block_rows, n), lambda i: (i, 0)),
        out_shape=jax.ShapeDtypeStruct((m, n), jnp.float32),
    )(a, w, b.reshape(1, n))


def _gatv2(x, ei, ea, Wl, bl, Wr, br, We, att, bias, heads, cout):
    n = x.shape[0]
    i, j = ei[0], ei[1]
    x_l = _mm(x, Wl, bl).reshape(n, heads, cout)
    x_r = _mm(x, Wr, br).reshape(n, heads, cout)
    e = _mm(ea, We, jnp.zeros((We.shape[1],), jnp.float32)).reshape(-1, heads, cout)
    m = x_l[j] + x_r[i] + e
    m = jax.nn.leaky_relu(m, negative_slope=0.2)
    alpha = jnp.sum(m * att, axis=-1)
    amax = jax.ops.segment_max(alpha, i, num_segments=n)
    alpha = jnp.exp(alpha - amax[i])
    asum = jax.ops.segment_sum(alpha, i, num_segments=n)
    alpha = alpha / (asum[i] + 1e-16)
    out = jax.ops.segment_sum(x_l[j] * alpha[:, :, None], i, num_segments=n)
    out = out.reshape(n, heads * cout) + bias
    return out, alpha


def kernel(x, edge_index, edge_attr, batch, Wl1, bl1, Wr1, br1, We1, att1, bias1,
           Wl2, bl2, Wr2, br2, We2, att2, bias2, ln_gamma, ln_beta):
    n = x.shape[0]
    dst = edge_index[1]
    deg = jax.ops.segment_sum(jnp.ones_like(dst, dtype=jnp.float32), dst, num_segments=n)
    loop_attr = jax.ops.segment_sum(edge_attr, dst, num_segments=n) / jnp.clip(deg, 1.0)[:, None]
    loops = jnp.arange(n, dtype=edge_index.dtype)
    ei = jnp.concatenate([edge_index, jnp.stack([loops, loops])], axis=1)
    ea = jnp.concatenate([edge_attr, loop_attr], axis=0)

    x1, a1 = _gatv2(x, ei, ea, Wl1, bl1, Wr1, br1, We1, att1, bias1, 1, C_OUT)
    x1 = jax.nn.leaky_relu(x1, negative_slope=0.01)
    x2, a2 = _gatv2(x1, ei, ea, Wl2, bl2, Wr2, br2, We2, att2, bias2, 1, C_OUT)
    x2 = jax.nn.leaky_relu(x2, negative_slope=0.01)
    pooled = jax.ops.segment_max(x2, batch, num_segments=NUM_GRAPHS)
    mean = jnp.mean(pooled, axis=-1, keepdims=True)
    var = jnp.var(pooled, axis=-1, keepdims=True)
    pooled = (pooled - mean) / jnp.sqrt(var + 1e-5) * ln_gamma + ln_beta
    return (pooled, ei, a1, a2)
